# Initial kernel scaffold; baseline (speedup 1.0000x reference)
import jax
import jax.numpy as jnp
from jax.experimental import pallas as pl


def kernel(x, table, W, b):
    raise NotImplementedError("write your pallas kernel here")


# speedup vs baseline: 1.0884x; 1.0884x over previous
import jax
import jax.numpy as jnp
from jax.experimental import pallas as pl
from jax.experimental.pallas import tpu as pltpu
from jax.experimental.pallas import tpu_sc as plsc

D = 64
GATHER_WINDOW = 128
TC_BLOCK = 8192


def _sc_gather(table, idx_flat):
    n = idx_flat.shape[1]
    mesh = plsc.VectorSubcoreMesh(core_axis_name="core",
                                  subcore_axis_name="subcore")

    @pl.kernel(
        out_type=jax.ShapeDtypeStruct((n, D), table.dtype),
        mesh=mesh,
        compiler_params=pltpu.CompilerParams(use_tc_tiling_on_sc=False),
    )
    def gather_kernel(table_hbm, idx_hbm, out_hbm):
        def body(idx_vmem, out_vmem):
            pltpu.sync_copy(table_hbm.at[idx_vmem.at[0]], out_vmem)

        pltpu.emit_pipeline(
            body,
            grid=(n // GATHER_WINDOW,),
            in_specs=[pl.BlockSpec((1, GATHER_WINDOW),
                                   index_map=lambda i: (0, i))],
            out_specs=[pl.BlockSpec((GATHER_WINDOW, D),
                                    index_map=lambda i: (i, 0))],
            core_axis_name=("core", "subcore"),
            dimension_semantics=(pltpu.PARALLEL,),
        )(idx_hbm, out_hbm)

    return gather_kernel(table, idx_flat)


def _tc_linear_tanh(emb, W, b):
    n = emb.shape[0]

    def body(emb_ref, w_ref, b_ref, o_ref):
        y = jax.lax.dot_general(
            emb_ref[...], w_ref[...],
            dimension_numbers=(((1,), (1,)), ((), ())),
            precision=jax.lax.Precision.HIGHEST,
            preferred_element_type=jnp.float32,
        )
        o_ref[...] = jnp.tanh(y + b_ref[...])

    return pl.pallas_call(
        body,
        grid=(n // TC_BLOCK,),
        in_specs=[
            pl.BlockSpec((TC_BLOCK, D), lambda i: (i, 0)),
            pl.BlockSpec((D, D), lambda i: (0, 0)),
            pl.BlockSpec((1, D), lambda i: (0, 0)),
        ],
        out_specs=pl.BlockSpec((TC_BLOCK, D), lambda i: (i, 0)),
        out_shape=jax.ShapeDtypeStruct((n, D), jnp.float32),
    )(emb, W, b.reshape(1, D))


def kernel(x, table, W, b):
    B, L = x.shape
    n = B * L
    idx_flat = x.reshape(1, n).astype(jnp.int32)
    emb = _sc_gather(table, idx_flat)
    out = _tc_linear_tanh(emb, W, b)
    return out.reshape(B, L, D)


# speedup vs baseline: 1.5951x; 1.4655x over previous
import jax
import jax.numpy as jnp
from jax.experimental import pallas as pl
from jax.experimental.pallas import tpu as pltpu
from jax.experimental.pallas import tpu_sc as plsc

D = 64
GATHER_WINDOW = 256
TC_BLOCK = 4096


def _sc_gather(table, idx_flat):
    n = idx_flat.shape[1]
    mesh = plsc.VectorSubcoreMesh(core_axis_name="core",
                                  subcore_axis_name="subcore")

    @pl.kernel(
        out_type=jax.ShapeDtypeStruct((n, D), table.dtype),
        mesh=mesh,
        compiler_params=pltpu.CompilerParams(use_tc_tiling_on_sc=False),
    )
    def gather_kernel(table_hbm, idx_hbm, out_hbm):
        def body(idx_vmem, out_vmem):
            pltpu.sync_copy(table_hbm.at[idx_vmem.at[0]], out_vmem)

        pltpu.emit_pipeline(
            body,
            grid=(n // GATHER_WINDOW,),
            in_specs=[pl.BlockSpec((1, GATHER_WINDOW),
                                   index_map=lambda i: (0, i))],
            out_specs=[pl.BlockSpec((GATHER_WINDOW, D),
                                    index_map=lambda i: (i, 0))],
            core_axis_name=("core", "subcore"),
            dimension_semantics=(pltpu.PARALLEL,),
        )(idx_hbm, out_hbm)

    return gather_kernel(table, idx_flat)


def _tc_linear_tanh(emb2, W2, b2):
    n2 = emb2.shape[0]

    def body(emb_ref, w_ref, b_ref, o_ref):
        y = jax.lax.dot_general(
            emb_ref[...], w_ref[...],
            dimension_numbers=(((1,), (0,)), ((), ())),
            preferred_element_type=jnp.float32,
        )
        o_ref[...] = jnp.tanh(y + b_ref[...])

    return pl.pallas_call(
        body,
        grid=(n2 // TC_BLOCK,),
        in_specs=[
            pl.BlockSpec((TC_BLOCK, 2 * D), lambda i: (i, 0)),
            pl.BlockSpec((2 * D, 2 * D), lambda i: (0, 0)),
            pl.BlockSpec((1, 2 * D), lambda i: (0, 0)),
        ],
        out_specs=pl.BlockSpec((TC_BLOCK, 2 * D), lambda i: (i, 0)),
        out_shape=jax.ShapeDtypeStruct((n2, 2 * D), jnp.float32),
    )(emb2, W2, b2)


def kernel(x, table, W, b):
    B, L = x.shape
    n = B * L
    idx_flat = x.reshape(1, n).astype(jnp.int32)
    emb = _sc_gather(table, idx_flat)
    emb2 = emb.reshape(n // 2, 2 * D)
    Wt = W.T
    W2 = jnp.zeros((2 * D, 2 * D), W.dtype).at[:D, :D].set(Wt).at[D:, D:].set(Wt)
    b2 = jnp.concatenate([b, b]).reshape(1, 2 * D)
    out2 = _tc_linear_tanh(emb2, W2, b2)
    return out2.reshape(B, L, D)
